# baseline (speedup 1.0000x reference)
import functools

import jax
import jax.numpy as jnp
from jax import lax
from jax.experimental import pallas as pl
from jax.experimental.pallas import tpu as pltpu
from jax.experimental.pallas import tpu_sc as plsc

N = 10000
E = 320000
D = 128
H = 128

_NC, _NS, _L = 2, 16, 16
_NW = _NC * _NS
_R = 320
_NPAD = _NW * _R
_SENT = _R
_CH = 3200
_NCHUNK = E // _CH
_CPL = _CH // _L
_G = 128
_CAP = _CH + _G + _L


def _l2n(h):
    n = jnp.sqrt(jnp.sum(h * h, axis=1, keepdims=True))
    return h / jnp.maximum(n, 1e-12)


def _bn(h, g, b):
    mu = jnp.mean(h, axis=0, keepdims=True)
    xc = h - mu
    var = jnp.mean(xc * xc, axis=0, keepdims=True)
    return xc * (g * lax.rsqrt(var + 1e-5)) + b


def _stage_a_body(x_ref, wp_ref, bp_ref, m_ref):
    m_ref[...] = jnp.maximum(
        jnp.dot(x_ref[...], wp_ref[...], preferred_element_type=jnp.float32)
        + bp_ref[...], 0.0)


def _stage_b_body(x_ref, agg_ref, ws_ref, wn_ref, b_ref, g0_ref, be0_ref,
                  wh_ref, bh_ref, gh_ref, beh_ref, wp1_ref, bp1_ref,
                  hh_ref, m1_ref):
    h = (jnp.dot(x_ref[...], ws_ref[...], preferred_element_type=jnp.float32)
         + jnp.dot(agg_ref[...], wn_ref[...], preferred_element_type=jnp.float32)
         + b_ref[...])
    h = _l2n(h)
    h = _bn(h, g0_ref[...], be0_ref[...])
    h = jnp.maximum(h, 0.0)
    hh = jnp.dot(h, wh_ref[...], preferred_element_type=jnp.float32) + bh_ref[...]
    hh = jnp.maximum(_bn(hh, gh_ref[...], beh_ref[...]), 0.0)
    hh_ref[...] = hh
    m1_ref[...] = jnp.maximum(
        jnp.dot(hh, wp1_ref[...], preferred_element_type=jnp.float32)
        + bp1_ref[...], 0.0)


def _stage_c_body(hh_ref, agg_ref, ws_ref, wn_ref, b_ref, wl_ref, bl_ref,
                  gl_ref, bel_ref, out_ref):
    h = (jnp.dot(hh_ref[...], ws_ref[...], preferred_element_type=jnp.float32)
         + jnp.dot(agg_ref[...], wn_ref[...], preferred_element_type=jnp.float32)
         + b_ref[...])
    h = _l2n(h)
    h = jnp.dot(h, wl_ref[...], preferred_element_type=jnp.float32) + bl_ref[...]
    out_ref[...] = _bn(h, gl_ref[...], bel_ref[...])


_f32 = functools.partial(jax.ShapeDtypeStruct, dtype=jnp.float32)


def _stage_a(x, wp, bp):
    return pl.pallas_call(_stage_a_body, out_shape=_f32((N, H)))(
        x, wp, bp.reshape(1, H))


def _stage_b(x, agg, ws, wn, b, g0, be0, wh, bh, gh, beh, wp1, bp1):
    return pl.pallas_call(
        _stage_b_body, out_shape=(_f32((N, H)), _f32((N, H))))(
            x, agg, ws, wn, b.reshape(1, H), g0.reshape(1, H),
            be0.reshape(1, H), wh, bh.reshape(1, H), gh.reshape(1, H),
            beh.reshape(1, H), wp1, bp1.reshape(1, H))


def _stage_c(hh, agg, ws, wn, b, wl, bl, gl, bel):
    return pl.pallas_call(_stage_c_body, out_shape=_f32((N, H)))(
        hh, agg, ws, wn, b.reshape(1, H), wl, bl.reshape(1, H),
        gl.reshape(1, H), bel.reshape(1, H))


def _segmax_body(m_hbm, src_hbm, dst_hbm, out_hbm, agg_v,
                 dst0, dst1, src0, src1, buck, cd0, cd1, cs0, cs1,
                 rows0, rows1, si0, si1, sr0, sr1):
    wid = lax.axis_index("s") * _NC + lax.axis_index("c")
    lo = wid * _R

    dstb = (dst0, dst1)
    srcb = (src0, src1)
    cdst = (cd0, cd1)
    csrc = (cs0, cs1)
    rows = (rows0, rows1)
    sem_i = (si0, si1)
    sem_r = (sr0, sr1)

    zeros = jnp.zeros((_L,), jnp.float32)
    laneoff = lax.iota(jnp.int32, _L) * _CPL
    ones = jnp.full((_L,), 1, jnp.int32)
    zsi = jnp.zeros((_L,), jnp.int32)
    sent_d = jnp.full((_L,), _SENT, jnp.int32)
    sent_s = jnp.zeros((_L,), jnp.int32)

    @pl.loop(0, _R + 1)
    def _zero(r):
        for c in range(H // _L):
            agg_v[r, pl.ds(c * _L, _L)] = zeros

    def fire_idx(ch, par):
        e0 = (ch % _NCHUNK) * _CH
        pltpu.async_copy(dst_hbm.at[pl.ds(e0, _CH)], dstb[par], sem_i[par])
        pltpu.async_copy(src_hbm.at[pl.ds(e0, _CH)], srcb[par], sem_i[par])

    def wait_idx(par):
        pltpu.make_async_copy(dst_hbm.at[pl.ds(0, _CH)], dstb[par],
                              sem_i[par]).wait()
        pltpu.make_async_copy(src_hbm.at[pl.ds(0, _CH)], srcb[par],
                              sem_i[par]).wait()

    def process_prev(par_prev, cnt_prev):
        nb = (cnt_prev + _G - 1) // _G

        @pl.when(cnt_prev > 0)
        def _():
            pltpu.make_async_copy(m_hbm.at[csrc[par_prev].at[pl.ds(0, _G)]],
                                  rows[par_prev], sem_r[par_prev]).wait()

            def grp0(gi, c2):
                dvec16 = cdst[par_prev][pl.ds(gi * _L, _L)]
                rbase = gi * _L
                for jj in range(_L):
                    d = dvec16[jj]
                    vals = []
                    for c in range(H // _L):
                        sl = pl.ds(c * _L, _L)
                        vals.append(jnp.maximum(agg_v[d, sl],
                                                rows[par_prev][rbase + jj,
                                                               sl]))
                    for c in range(H // _L):
                        agg_v[d, pl.ds(c * _L, _L)] = vals[c]
                return c2

            lax.fori_loop(0, _G // _L, grp0, 0)

            def ovf(j, c2):
                base = j * _G
                pltpu.async_copy(
                    m_hbm.at[csrc[par_prev].at[pl.ds(base, _G)]],
                    rows[par_prev], sem_r[par_prev]).wait()

                def grp(gi, c3):
                    dvec16 = cdst[par_prev][pl.ds(base + gi * _L, _L)]
                    rbase = gi * _L
                    for jj in range(_L):
                        d = dvec16[jj]
                        vals = []
                        for c in range(H // _L):
                            sl = pl.ds(c * _L, _L)
                            vals.append(jnp.maximum(agg_v[d, sl],
                                                    rows[par_prev][rbase + jj,
                                                                   sl]))
                        for c in range(H // _L):
                            agg_v[d, pl.ds(c * _L, _L)] = vals[c]
                    return c3

                lax.fori_loop(0, _G // _L, grp, 0)
                return c2

            lax.fori_loop(1, nb, ovf, 0)

    def chunk_step(ch, par, cnt_prev):
        fire_idx(ch + 1, 1 - par)
        with jax.named_scope("idxwait"):
            wait_idx(par)

        def scan_body(g, cntv):
            dvec = dstb[par][pl.ds(g * _L, _L)]
            svec = srcb[par][pl.ds(g * _L, _L)]
            msk = (dvec >= lo) & (dvec < lo + _R)
            val = jnp.left_shift(dvec - lo, 14) | svec
            plsc.store_scatter(buck, [laneoff + cntv], val, mask=msk)
            return cntv + jnp.where(msk, ones, zsi)

        with jax.named_scope("scan"):
            cntv = lax.fori_loop(0, _CH // _L, scan_body, zsi)

        _ms = jax.named_scope("merge"); _ms.__enter__()
        tot = 0
        for l in range(_L):
            c_l = cntv[l]

            def mv(k, t):
                v = buck[pl.ds(l * _CPL + k * _L, _L)]
                csrc[par][pl.ds(t + k * _L, _L)] = v & 16383
                cdst[par][pl.ds(t + k * _L, _L)] = jnp.right_shift(v, 14)
                return t

            lax.fori_loop(0, (c_l + _L - 1) // _L,
                          functools.partial(mv), tot)
            tot = tot + c_l

        for t in range(_G // _L):
            cdst[par][pl.ds(tot + t * _L, _L)] = sent_d
            csrc[par][pl.ds(tot + t * _L, _L)] = sent_s

        _ms.__exit__(None, None, None)
        @pl.when(tot > 0)
        def _():
            pltpu.async_copy(m_hbm.at[csrc[par].at[pl.ds(0, _G)]],
                             rows[par], sem_r[par])

        with jax.named_scope("maxacc"):
            process_prev(1 - par, cnt_prev)
        return tot

    fire_idx(0, 0)

    def pair_body(chp, cnt_prev):
        c0 = chunk_step(2 * chp, 0, cnt_prev)
        c1 = chunk_step(2 * chp + 1, 1, c0)
        return c1

    cnt_last = lax.fori_loop(0, _NCHUNK // 2, pair_body, 0)

    wait_idx(0)
    process_prev(1, cnt_last)

    pltpu.sync_copy(agg_v.at[pl.ds(0, _R)], out_hbm.at[pl.ds(lo, _R)])


_segmax_call = pl.kernel(
    _segmax_body,
    out_type=jax.ShapeDtypeStruct((_NPAD, H), jnp.float32),
    mesh=plsc.VectorSubcoreMesh(core_axis_name="c", subcore_axis_name="s",
                                num_cores=_NC, num_subcores=_NS),
    compiler_params=pltpu.CompilerParams(needs_layout_passes=False),
    scratch_types=[
        pltpu.VMEM((_R + 1, H), jnp.float32),
        pltpu.VMEM((_CH,), jnp.int32),
        pltpu.VMEM((_CH,), jnp.int32),
        pltpu.VMEM((_CH,), jnp.int32),
        pltpu.VMEM((_CH,), jnp.int32),
        pltpu.VMEM((_CH,), jnp.int32),
        pltpu.VMEM((_CAP,), jnp.int32),
        pltpu.VMEM((_CAP,), jnp.int32),
        pltpu.VMEM((_CAP,), jnp.int32),
        pltpu.VMEM((_CAP,), jnp.int32),
        pltpu.VMEM((_G, H), jnp.float32),
        pltpu.VMEM((_G, H), jnp.float32),
        pltpu.SemaphoreType.DMA,
        pltpu.SemaphoreType.DMA,
        pltpu.SemaphoreType.DMA,
        pltpu.SemaphoreType.DMA,
    ],
)


def _segmax(m, src, dst):
    return _segmax_call(m, src, dst)[:N]


def kernel(x, edge_index0, edge_index1, Wp0, bp0, Ws0, Wn0, b0, Wp1, bp1,
           Ws1, Wn1, b1, g_bn0, be_bn0, Wh, bh, g_h, be_h, Wl, bl, g_l,
           be_l):
    m0 = _stage_a(x, Wp0, bp0)
    agg0 = _segmax(m0, edge_index0[0], edge_index0[1])
    hh, m1 = _stage_b(x, agg0, Ws0, Wn0, b0, g_bn0, be_bn0, Wh, bh, g_h,
                      be_h, Wp1, bp1)
    agg1 = _segmax(m1, edge_index1[0], edge_index1[1])
    return _stage_c(hh, agg1, Ws1, Wn1, b1, Wl, bl, g_l, be_l)


# speedup vs baseline: 4.4683x; 4.4683x over previous
import functools

import jax
import jax.numpy as jnp
from jax import lax
from jax.experimental import pallas as pl
from jax.experimental.pallas import tpu as pltpu
from jax.experimental.pallas import tpu_sc as plsc

N = 10000
E = 320000
D = 128
H = 128

_NC, _NS, _L = 2, 16, 16
_NW = _NC * _NS
_R = 320
_NPAD = _NW * _R
_SENT = _R
_CH = 3200
_NCHUNK = E // _CH
_CW = 4
_NP = 10240


def _l2n(h):
    n = jnp.sqrt(jnp.sum(h * h, axis=1, keepdims=True))
    return h / jnp.maximum(n, 1e-12)


def _bn(h, g, b):
    mu = jnp.mean(h, axis=0, keepdims=True)
    xc = h - mu
    var = jnp.mean(xc * xc, axis=0, keepdims=True)
    return xc * (g * lax.rsqrt(var + 1e-5)) + b


def _stage_a_body(x_ref, wp_ref, bp_ref, m_ref):
    m_ref[...] = jnp.maximum(
        jnp.dot(x_ref[...], wp_ref[...], preferred_element_type=jnp.float32)
        + bp_ref[...], 0.0)


def _stage_b_body(x_ref, agg_ref, ws_ref, wn_ref, b_ref, g0_ref, be0_ref,
                  wh_ref, bh_ref, gh_ref, beh_ref, wp1_ref, bp1_ref,
                  hh_ref, m1_ref):
    h = (jnp.dot(x_ref[...], ws_ref[...], preferred_element_type=jnp.float32)
         + jnp.dot(agg_ref[...], wn_ref[...], preferred_element_type=jnp.float32)
         + b_ref[...])
    h = _l2n(h)
    h = _bn(h, g0_ref[...], be0_ref[...])
    h = jnp.maximum(h, 0.0)
    hh = jnp.dot(h, wh_ref[...], preferred_element_type=jnp.float32) + bh_ref[...]
    hh = jnp.maximum(_bn(hh, gh_ref[...], beh_ref[...]), 0.0)
    hh_ref[...] = hh
    m1_ref[...] = jnp.maximum(
        jnp.dot(hh, wp1_ref[...], preferred_element_type=jnp.float32)
        + bp1_ref[...], 0.0)


def _stage_c_body(hh_ref, agg_ref, ws_ref, wn_ref, b_ref, wl_ref, bl_ref,
                  gl_ref, bel_ref, out_ref):
    h = (jnp.dot(hh_ref[...], ws_ref[...], preferred_element_type=jnp.float32)
         + jnp.dot(agg_ref[...], wn_ref[...], preferred_element_type=jnp.float32)
         + b_ref[...])
    h = _l2n(h)
    h = jnp.dot(h, wl_ref[...], preferred_element_type=jnp.float32) + bl_ref[...]
    out_ref[...] = _bn(h, gl_ref[...], bel_ref[...])


_f32 = functools.partial(jax.ShapeDtypeStruct, dtype=jnp.float32)


def _stage_a(x, wp, bp):
    return pl.pallas_call(_stage_a_body, out_shape=_f32((N, H)))(
        x, wp, bp.reshape(1, H))


def _stage_b(x, agg, ws, wn, b, g0, be0, wh, bh, gh, beh, wp1, bp1):
    return pl.pallas_call(
        _stage_b_body, out_shape=(_f32((N, H)), _f32((N, H))))(
            x, agg, ws, wn, b.reshape(1, H), g0.reshape(1, H),
            be0.reshape(1, H), wh, bh.reshape(1, H), gh.reshape(1, H),
            beh.reshape(1, H), wp1, bp1.reshape(1, H))


def _stage_c(hh, agg, ws, wn, b, wl, bl, gl, bel):
    return pl.pallas_call(_stage_c_body, out_shape=_f32((N, H)))(
        hh, agg, ws, wn, b.reshape(1, H), wl, bl.reshape(1, H),
        gl.reshape(1, H), bel.reshape(1, H))


def _segmax_body(mt_hbm, src_hbm, dst_hbm, out_hbm,
                 mloc, agg_v, tmp, dst0, dst1, src0, src1, si0, si1):
    wid = lax.axis_index("s") * _NC + lax.axis_index("c")

    dstb = (dst0, dst1)
    srcb = (src0, src1)
    sem_i = (si0, si1)

    zeros = jnp.zeros((_L,), jnp.float32)
    lane = lax.iota(jnp.int32, _L)

    pltpu.sync_copy(mt_hbm.at[wid], mloc)

    @pl.loop(0, _NP // _L)
    def _zero(r):
        for c in range(_CW):
            agg_v[c, pl.ds(r * _L, _L)] = zeros

    def fire_idx(ch, par):
        e0 = (ch % _NCHUNK) * _CH
        pltpu.async_copy(dst_hbm.at[pl.ds(e0, _CH)], dstb[par], sem_i[par])
        pltpu.async_copy(src_hbm.at[pl.ds(e0, _CH)], srcb[par], sem_i[par])

    def wait_idx(par):
        pltpu.make_async_copy(dst_hbm.at[pl.ds(0, _CH)], dstb[par],
                              sem_i[par]).wait()
        pltpu.make_async_copy(src_hbm.at[pl.ds(0, _CH)], srcb[par],
                              sem_i[par]).wait()

    def chunk_step(ch, par):
        fire_idx(ch + 1, 1 - par)
        wait_idx(par)

        def scan_body(g, carry):
            dvec = dstb[par][pl.ds(g * _L, _L)]
            svec = srcb[par][pl.ds(g * _L, _L)]
            plsc.store_scatter(tmp, [dvec], lane)
            rd = plsc.load_gather(tmp, [dvec])
            dup = rd != lane
            nodup = rd == lane
            vals = []
            for c in range(_CW):
                cc = jnp.full((_L,), c, jnp.int32)
                v = plsc.load_gather(mloc, [cc, svec])
                cur = plsc.load_gather(agg_v, [cc, dvec])
                plsc.store_scatter(agg_v, [cc, dvec],
                                   jnp.maximum(cur, v), mask=nodup)
                vals.append(v)
            ndup = plsc.all_reduce_population_count(dup)

            @pl.when(ndup[0] > 0)
            def _():
                active = dup
                for _r in range(_L - 1):
                    done = jnp.full((_L,), True, jnp.bool_)
                    for c in range(_CW):
                        cc = jnp.full((_L,), c, jnp.int32)
                        cur = plsc.load_gather(agg_v, [cc, dvec])
                        plsc.store_scatter(agg_v, [cc, dvec],
                                           jnp.maximum(cur, vals[c]),
                                           mask=active)
                    for c in range(_CW):
                        cc = jnp.full((_L,), c, jnp.int32)
                        chk = plsc.load_gather(agg_v, [cc, dvec])
                        done = done & (chk >= vals[c])
                    active = active & (~done)
            return carry

        lax.fori_loop(0, _CH // _L, scan_body, 0)

    fire_idx(0, 0)

    def pair_body(chp, carry):
        chunk_step(2 * chp, 0)
        chunk_step(2 * chp + 1, 1)
        return carry

    lax.fori_loop(0, _NCHUNK // 2, pair_body, 0)

    wait_idx(0)
    pltpu.sync_copy(agg_v, out_hbm.at[wid])


_segmax_call = pl.kernel(
    _segmax_body,
    out_type=jax.ShapeDtypeStruct((_NW, _CW, _NP), jnp.float32),
    mesh=plsc.VectorSubcoreMesh(core_axis_name="c", subcore_axis_name="s",
                                num_cores=_NC, num_subcores=_NS),
    compiler_params=pltpu.CompilerParams(needs_layout_passes=False),
    scratch_types=[
        pltpu.VMEM((_CW, N), jnp.float32),
        pltpu.VMEM((_CW, _NP), jnp.float32),
        pltpu.VMEM((_NP,), jnp.int32),
        pltpu.VMEM((_CH,), jnp.int32),
        pltpu.VMEM((_CH,), jnp.int32),
        pltpu.VMEM((_CH,), jnp.int32),
        pltpu.VMEM((_CH,), jnp.int32),
        pltpu.SemaphoreType.DMA,
        pltpu.SemaphoreType.DMA,
    ],
)


def _segmax(m, src, dst):
    mt = jnp.transpose(m.reshape(N, _NW, _CW), (1, 2, 0))
    out_t = _segmax_call(mt, src, dst)
    return jnp.transpose(out_t, (2, 0, 1)).reshape(_NP, H)[:N]


def kernel(x, edge_index0, edge_index1, Wp0, bp0, Ws0, Wn0, b0, Wp1, bp1,
           Ws1, Wn1, b1, g_bn0, be_bn0, Wh, bh, g_h, be_h, Wl, bl, g_l,
           be_l):
    m0 = _stage_a(x, Wp0, bp0)
    agg0 = _segmax(m0, edge_index0[0], edge_index0[1])
    hh, m1 = _stage_b(x, agg0, Ws0, Wn0, b0, g_bn0, be_bn0, Wh, bh, g_h,
                      be_h, Wp1, bp1)
    agg1 = _segmax(m1, edge_index1[0], edge_index1[1])
    return _stage_c(hh, agg1, Ws1, Wn1, b1, Wl, bl, g_l, be_l)


# speedup vs baseline: 4.6331x; 1.0369x over previous
import functools

import jax
import jax.numpy as jnp
from jax import lax
from jax.experimental import pallas as pl
from jax.experimental.pallas import tpu as pltpu
from jax.experimental.pallas import tpu_sc as plsc

N = 10000
E = 320000
D = 128
H = 128

_NC, _NS, _L = 2, 16, 16
_NW = _NC * _NS
_R = 320
_NPAD = _NW * _R
_SENT = _R
_CH = 6400
_NCHUNK = E // _CH
_CW = 4
_NP = 10240


def _l2n(h):
    n = jnp.sqrt(jnp.sum(h * h, axis=1, keepdims=True))
    return h / jnp.maximum(n, 1e-12)


def _bn(h, g, b):
    mu = jnp.mean(h, axis=0, keepdims=True)
    xc = h - mu
    var = jnp.mean(xc * xc, axis=0, keepdims=True)
    return xc * (g * lax.rsqrt(var + 1e-5)) + b


def _stage_a_body(x_ref, wp_ref, bp_ref, m_ref):
    m_ref[...] = jnp.maximum(
        jnp.dot(x_ref[...], wp_ref[...], preferred_element_type=jnp.float32)
        + bp_ref[...], 0.0)


def _stage_b_body(x_ref, agg_ref, ws_ref, wn_ref, b_ref, g0_ref, be0_ref,
                  wh_ref, bh_ref, gh_ref, beh_ref, wp1_ref, bp1_ref,
                  hh_ref, m1_ref):
    h = (jnp.dot(x_ref[...], ws_ref[...], preferred_element_type=jnp.float32)
         + jnp.dot(agg_ref[...], wn_ref[...], preferred_element_type=jnp.float32)
         + b_ref[...])
    h = _l2n(h)
    h = _bn(h, g0_ref[...], be0_ref[...])
    h = jnp.maximum(h, 0.0)
    hh = jnp.dot(h, wh_ref[...], preferred_element_type=jnp.float32) + bh_ref[...]
    hh = jnp.maximum(_bn(hh, gh_ref[...], beh_ref[...]), 0.0)
    hh_ref[...] = hh
    m1_ref[...] = jnp.maximum(
        jnp.dot(hh, wp1_ref[...], preferred_element_type=jnp.float32)
        + bp1_ref[...], 0.0)


def _stage_c_body(hh_ref, agg_ref, ws_ref, wn_ref, b_ref, wl_ref, bl_ref,
                  gl_ref, bel_ref, out_ref):
    h = (jnp.dot(hh_ref[...], ws_ref[...], preferred_element_type=jnp.float32)
         + jnp.dot(agg_ref[...], wn_ref[...], preferred_element_type=jnp.float32)
         + b_ref[...])
    h = _l2n(h)
    h = jnp.dot(h, wl_ref[...], preferred_element_type=jnp.float32) + bl_ref[...]
    out_ref[...] = _bn(h, gl_ref[...], bel_ref[...])


_f32 = functools.partial(jax.ShapeDtypeStruct, dtype=jnp.float32)


def _stage_a(x, wp, bp):
    return pl.pallas_call(_stage_a_body, out_shape=_f32((N, H)))(
        x, wp, bp.reshape(1, H))


def _stage_b(x, agg, ws, wn, b, g0, be0, wh, bh, gh, beh, wp1, bp1):
    return pl.pallas_call(
        _stage_b_body, out_shape=(_f32((N, H)), _f32((N, H))))(
            x, agg, ws, wn, b.reshape(1, H), g0.reshape(1, H),
            be0.reshape(1, H), wh, bh.reshape(1, H), gh.reshape(1, H),
            beh.reshape(1, H), wp1, bp1.reshape(1, H))


def _stage_c(hh, agg, ws, wn, b, wl, bl, gl, bel):
    return pl.pallas_call(_stage_c_body, out_shape=_f32((N, H)))(
        hh, agg, ws, wn, b.reshape(1, H), wl, bl.reshape(1, H),
        gl.reshape(1, H), bel.reshape(1, H))


def _segmax_body(mt_hbm, src_hbm, dst_hbm, out_hbm,
                 mloc, agg_v, tmp, dst0, dst1, src0, src1, si0, si1):
    wid = lax.axis_index("s") * _NC + lax.axis_index("c")

    dstb = (dst0, dst1)
    srcb = (src0, src1)
    sem_i = (si0, si1)

    zeros = jnp.zeros((_L,), jnp.float32)
    lane = lax.iota(jnp.int32, _L)

    pltpu.sync_copy(mt_hbm.at[wid], mloc)

    @pl.loop(0, _NP // _L)
    def _zero(r):
        for c in range(_CW):
            agg_v[c, pl.ds(r * _L, _L)] = zeros

    def fire_idx(ch, par):
        e0 = (ch % _NCHUNK) * _CH
        pltpu.async_copy(dst_hbm.at[pl.ds(e0, _CH)], dstb[par], sem_i[par])
        pltpu.async_copy(src_hbm.at[pl.ds(e0, _CH)], srcb[par], sem_i[par])

    def wait_idx(par):
        pltpu.make_async_copy(dst_hbm.at[pl.ds(0, _CH)], dstb[par],
                              sem_i[par]).wait()
        pltpu.make_async_copy(src_hbm.at[pl.ds(0, _CH)], srcb[par],
                              sem_i[par]).wait()

    def chunk_step(ch, par):
        fire_idx(ch + 1, 1 - par)
        wait_idx(par)

        def scan_body(g, carry):
            dvec = dstb[par][pl.ds(g * _L, _L)]
            svec = srcb[par][pl.ds(g * _L, _L)]
            plsc.store_scatter(tmp, [dvec], lane)
            rd = plsc.load_gather(tmp, [dvec])
            dup = rd != lane
            nodup = rd == lane
            vals = []
            for c in range(_CW):
                cc = jnp.full((_L,), c, jnp.int32)
                v = plsc.load_gather(mloc, [cc, svec])
                cur = plsc.load_gather(agg_v, [cc, dvec])
                plsc.store_scatter(agg_v, [cc, dvec],
                                   jnp.maximum(cur, v), mask=nodup)
                vals.append(v)
            ndup = plsc.all_reduce_population_count(dup)

            @pl.when(ndup[0] > 0)
            def _():
                active = dup
                for _r in range(_L - 1):
                    done = jnp.full((_L,), True, jnp.bool_)
                    for c in range(_CW):
                        cc = jnp.full((_L,), c, jnp.int32)
                        cur = plsc.load_gather(agg_v, [cc, dvec])
                        plsc.store_scatter(agg_v, [cc, dvec],
                                           jnp.maximum(cur, vals[c]),
                                           mask=active)
                    for c in range(_CW):
                        cc = jnp.full((_L,), c, jnp.int32)
                        chk = plsc.load_gather(agg_v, [cc, dvec])
                        done = done & (chk >= vals[c])
                    active = active & (~done)
            return carry

        lax.fori_loop(0, _CH // _L, scan_body, 0, unroll=2)

    fire_idx(0, 0)

    def pair_body(chp, carry):
        chunk_step(2 * chp, 0)
        chunk_step(2 * chp + 1, 1)
        return carry

    lax.fori_loop(0, _NCHUNK // 2, pair_body, 0)

    wait_idx(0)
    pltpu.sync_copy(agg_v, out_hbm.at[wid])


_segmax_call = pl.kernel(
    _segmax_body,
    out_type=jax.ShapeDtypeStruct((_NW, _CW, _NP), jnp.float32),
    mesh=plsc.VectorSubcoreMesh(core_axis_name="c", subcore_axis_name="s",
                                num_cores=_NC, num_subcores=_NS),
    compiler_params=pltpu.CompilerParams(needs_layout_passes=False),
    scratch_types=[
        pltpu.VMEM((_CW, N), jnp.float32),
        pltpu.VMEM((_CW, _NP), jnp.float32),
        pltpu.VMEM((_NP,), jnp.int32),
        pltpu.VMEM((_CH,), jnp.int32),
        pltpu.VMEM((_CH,), jnp.int32),
        pltpu.VMEM((_CH,), jnp.int32),
        pltpu.VMEM((_CH,), jnp.int32),
        pltpu.SemaphoreType.DMA,
        pltpu.SemaphoreType.DMA,
    ],
)


def _segmax(m, src, dst):
    mt = jnp.transpose(m.reshape(N, _NW, _CW), (1, 2, 0))
    out_t = _segmax_call(mt, src, dst)
    return jnp.transpose(out_t, (2, 0, 1)).reshape(_NP, H)[:N]


def kernel(x, edge_index0, edge_index1, Wp0, bp0, Ws0, Wn0, b0, Wp1, bp1,
           Ws1, Wn1, b1, g_bn0, be_bn0, Wh, bh, g_h, be_h, Wl, bl, g_l,
           be_l):
    m0 = _stage_a(x, Wp0, bp0)
    agg0 = _segmax(m0, edge_index0[0], edge_index0[1])
    hh, m1 = _stage_b(x, agg0, Ws0, Wn0, b0, g_bn0, be_bn0, Wh, bh, g_h,
                      be_h, Wp1, bp1)
    agg1 = _segmax(m1, edge_index1[0], edge_index1[1])
    return _stage_c(hh, agg1, Ws1, Wn1, b1, Wl, bl, g_l, be_l)


# speedup vs baseline: 4.6547x; 1.0047x over previous
import functools

import jax
import jax.numpy as jnp
from jax import lax
from jax.experimental import pallas as pl
from jax.experimental.pallas import tpu as pltpu
from jax.experimental.pallas import tpu_sc as plsc

N = 10000
E = 320000
D = 128
H = 128

_NC, _NS, _L = 2, 16, 16
_NW = _NC * _NS
_R = 320
_NPAD = _NW * _R
_SENT = _R
_CH = 8000
_NCHUNK = E // _CH
_CW = 4
_NP = 10240


def _l2n(h):
    n = jnp.sqrt(jnp.sum(h * h, axis=1, keepdims=True))
    return h / jnp.maximum(n, 1e-12)


def _bn(h, g, b):
    mu = jnp.mean(h, axis=0, keepdims=True)
    xc = h - mu
    var = jnp.mean(xc * xc, axis=0, keepdims=True)
    return xc * (g * lax.rsqrt(var + 1e-5)) + b


def _stage_a_body(x_ref, wp_ref, bp_ref, m_ref):
    m_ref[...] = jnp.maximum(
        jnp.dot(x_ref[...], wp_ref[...], preferred_element_type=jnp.float32)
        + bp_ref[...], 0.0)


def _stage_b_body(x_ref, agg_ref, ws_ref, wn_ref, b_ref, g0_ref, be0_ref,
                  wh_ref, bh_ref, gh_ref, beh_ref, wp1_ref, bp1_ref,
                  hh_ref, m1_ref):
    h = (jnp.dot(x_ref[...], ws_ref[...], preferred_element_type=jnp.float32)
         + jnp.dot(agg_ref[...], wn_ref[...], preferred_element_type=jnp.float32)
         + b_ref[...])
    h = _l2n(h)
    h = _bn(h, g0_ref[...], be0_ref[...])
    h = jnp.maximum(h, 0.0)
    hh = jnp.dot(h, wh_ref[...], preferred_element_type=jnp.float32) + bh_ref[...]
    hh = jnp.maximum(_bn(hh, gh_ref[...], beh_ref[...]), 0.0)
    hh_ref[...] = hh
    m1_ref[...] = jnp.maximum(
        jnp.dot(hh, wp1_ref[...], preferred_element_type=jnp.float32)
        + bp1_ref[...], 0.0)


def _stage_c_body(hh_ref, agg_ref, ws_ref, wn_ref, b_ref, wl_ref, bl_ref,
                  gl_ref, bel_ref, out_ref):
    h = (jnp.dot(hh_ref[...], ws_ref[...], preferred_element_type=jnp.float32)
         + jnp.dot(agg_ref[...], wn_ref[...], preferred_element_type=jnp.float32)
         + b_ref[...])
    h = _l2n(h)
    h = jnp.dot(h, wl_ref[...], preferred_element_type=jnp.float32) + bl_ref[...]
    out_ref[...] = _bn(h, gl_ref[...], bel_ref[...])


_f32 = functools.partial(jax.ShapeDtypeStruct, dtype=jnp.float32)


def _stage_a(x, wp, bp):
    return pl.pallas_call(_stage_a_body, out_shape=_f32((N, H)))(
        x, wp, bp.reshape(1, H))


def _stage_b(x, agg, ws, wn, b, g0, be0, wh, bh, gh, beh, wp1, bp1):
    return pl.pallas_call(
        _stage_b_body, out_shape=(_f32((N, H)), _f32((N, H))))(
            x, agg, ws, wn, b.reshape(1, H), g0.reshape(1, H),
            be0.reshape(1, H), wh, bh.reshape(1, H), gh.reshape(1, H),
            beh.reshape(1, H), wp1, bp1.reshape(1, H))


def _stage_c(hh, agg, ws, wn, b, wl, bl, gl, bel):
    return pl.pallas_call(_stage_c_body, out_shape=_f32((N, H)))(
        hh, agg, ws, wn, b.reshape(1, H), wl, bl.reshape(1, H),
        gl.reshape(1, H), bel.reshape(1, H))


def _segmax_body(mt_hbm, src_hbm, dst_hbm, out_hbm,
                 mloc, agg_v, tmp, dst0, dst1, src0, src1, si0, si1):
    wid = lax.axis_index("s") * _NC + lax.axis_index("c")

    dstb = (dst0, dst1)
    srcb = (src0, src1)
    sem_i = (si0, si1)

    zeros = jnp.zeros((_L,), jnp.float32)
    lane = lax.iota(jnp.int32, _L)

    pltpu.sync_copy(mt_hbm.at[wid], mloc)

    @pl.loop(0, _NP // _L)
    def _zero(r):
        for c in range(_CW):
            agg_v[c, pl.ds(r * _L, _L)] = zeros

    def fire_idx(ch, par):
        e0 = (ch % _NCHUNK) * _CH
        pltpu.async_copy(dst_hbm.at[pl.ds(e0, _CH)], dstb[par], sem_i[par])
        pltpu.async_copy(src_hbm.at[pl.ds(e0, _CH)], srcb[par], sem_i[par])

    def wait_idx(par):
        pltpu.make_async_copy(dst_hbm.at[pl.ds(0, _CH)], dstb[par],
                              sem_i[par]).wait()
        pltpu.make_async_copy(src_hbm.at[pl.ds(0, _CH)], srcb[par],
                              sem_i[par]).wait()

    def chunk_step(ch, par):
        fire_idx(ch + 1, 1 - par)
        wait_idx(par)

        def scan_body(g, carry):
            dvec = dstb[par][pl.ds(g * _L, _L)]
            svec = srcb[par][pl.ds(g * _L, _L)]
            plsc.store_scatter(tmp, [dvec], lane)
            rd = plsc.load_gather(tmp, [dvec])
            dup = rd != lane
            nodup = rd == lane
            vals = []
            for c in range(_CW):
                cc = jnp.full((_L,), c, jnp.int32)
                v = plsc.load_gather(mloc, [cc, svec])
                cur = plsc.load_gather(agg_v, [cc, dvec])
                plsc.store_scatter(agg_v, [cc, dvec],
                                   jnp.maximum(cur, v), mask=nodup)
                vals.append(v)
            ndup = plsc.all_reduce_population_count(dup)

            @pl.when(ndup[0] > 0)
            def _():
                active = dup
                for _r in range(_L - 1):
                    done = jnp.full((_L,), True, jnp.bool_)
                    for c in range(_CW):
                        cc = jnp.full((_L,), c, jnp.int32)
                        cur = plsc.load_gather(agg_v, [cc, dvec])
                        plsc.store_scatter(agg_v, [cc, dvec],
                                           jnp.maximum(cur, vals[c]),
                                           mask=active)
                    for c in range(_CW):
                        cc = jnp.full((_L,), c, jnp.int32)
                        chk = plsc.load_gather(agg_v, [cc, dvec])
                        done = done & (chk >= vals[c])
                    active = active & (~done)
            return carry

        lax.fori_loop(0, _CH // _L, scan_body, 0, unroll=4)

    fire_idx(0, 0)

    def pair_body(chp, carry):
        chunk_step(2 * chp, 0)
        chunk_step(2 * chp + 1, 1)
        return carry

    lax.fori_loop(0, _NCHUNK // 2, pair_body, 0)

    wait_idx(0)
    pltpu.sync_copy(agg_v, out_hbm.at[wid])


_segmax_call = pl.kernel(
    _segmax_body,
    out_type=jax.ShapeDtypeStruct((_NW, _CW, _NP), jnp.float32),
    mesh=plsc.VectorSubcoreMesh(core_axis_name="c", subcore_axis_name="s",
                                num_cores=_NC, num_subcores=_NS),
    compiler_params=pltpu.CompilerParams(needs_layout_passes=False),
    scratch_types=[
        pltpu.VMEM((_CW, N), jnp.float32),
        pltpu.VMEM((_CW, _NP), jnp.float32),
        pltpu.VMEM((_NP,), jnp.int32),
        pltpu.VMEM((_CH,), jnp.int32),
        pltpu.VMEM((_CH,), jnp.int32),
        pltpu.VMEM((_CH,), jnp.int32),
        pltpu.VMEM((_CH,), jnp.int32),
        pltpu.SemaphoreType.DMA,
        pltpu.SemaphoreType.DMA,
    ],
)


def _segmax(m, src, dst):
    mt = jnp.transpose(m.reshape(N, _NW, _CW), (1, 2, 0))
    out_t = _segmax_call(mt, src, dst)
    return jnp.transpose(out_t, (2, 0, 1)).reshape(_NP, H)[:N]


def kernel(x, edge_index0, edge_index1, Wp0, bp0, Ws0, Wn0, b0, Wp1, bp1,
           Ws1, Wn1, b1, g_bn0, be_bn0, Wh, bh, g_h, be_h, Wl, bl, g_l,
           be_l):
    m0 = _stage_a(x, Wp0, bp0)
    agg0 = _segmax(m0, edge_index0[0], edge_index0[1])
    hh, m1 = _stage_b(x, agg0, Ws0, Wn0, b0, g_bn0, be_bn0, Wh, bh, g_h,
                      be_h, Wp1, bp1)
    agg1 = _segmax(m1, edge_index1[0], edge_index1[1])
    return _stage_c(hh, agg1, Ws1, Wn1, b1, Wl, bl, g_l, be_l)
